# baseline (speedup 1.0000x reference)
import jax
import jax.numpy as jnp
from jax import lax
from jax.experimental import pallas as pl
from jax.experimental.pallas import tpu as pltpu
from jax.experimental.pallas import tpu_sc as plsc


def _field_sc(values_reduced, imposed_full, free_idx, conn_flat,
              n_nodes, n_free, n_flat):
  info = plsc.get_sparse_core_info()
  nc, ns = info.num_cores, info.num_subcores
  nw = nc * ns

  C1 = 896
  n1 = -(-n_nodes // C1)
  s1 = -(-n1 // ns)
  r1_last = n_nodes - C1

  nb2 = -(-n_free // 128)
  s2 = -(-nb2 // ns)
  off2_last = n_free - 128

  B3 = 2048
  per_w = -(-n_flat // nw)
  per_w = -(-per_w // B3) * B3
  base_last = n_flat - per_w
  steps = per_w // B3
  t_last = steps - 1
  steps_r = -(-steps // 2) * 2

  mesh = plsc.VectorSubcoreMesh(core_axis_name="c", subcore_axis_name="s")

  @pl.kernel(
      out_type=jax.ShapeDtypeStruct((n_flat, 3), jnp.float32),
      mesh=mesh,
      compiler_params=pltpu.CompilerParams(use_tc_tiling_on_sc=False),
      scratch_types=[
          pltpu.VMEM_SHARED((n_nodes, 8), jnp.float32),
          pltpu.VMEM((C1, 8), jnp.float32),
          pltpu.VMEM((1, 128), jnp.int32),
          pltpu.VMEM((128, 8), jnp.float32),
          [pltpu.VMEM((B3,), jnp.int32)] * 2,
          [pltpu.VMEM((B3, 8), jnp.float32)] * 2,
          [pltpu.SemaphoreType.DMA] * 2,
          [pltpu.SemaphoreType.DMA] * 2,
      ],
  )
  def body(vals_hbm, imp_hbm, free_hbm, conn_hbm, out_hbm,
           table, stage, sidx, svals, gidx, grows, si, so):
    cid = lax.axis_index("c")
    sid = lax.axis_index("s")
    wid = sid * nc + cid

    @pl.loop(0, s1)
    def _copy(i):
      c = sid + i * ns
      r0 = jnp.minimum(c * C1, r1_last)
      pltpu.sync_copy(imp_hbm.at[pl.ds(r0, C1)], stage.at[:, pl.ds(0, 3)])
      pltpu.sync_copy(stage, table.at[pl.ds(r0, C1)])
    plsc.subcore_barrier()

    @pl.loop(0, s2)
    def _scatter(k):
      b = sid + k * ns
      off = jnp.minimum(b * 128, off2_last)
      pltpu.sync_copy(free_hbm.at[pl.ds(off, 128)], sidx.at[0])
      pltpu.sync_copy(vals_hbm.at[pl.ds(off, 128)], svals.at[:, pl.ds(0, 3)])
      pltpu.sync_copy(svals, table.at[sidx.at[0]])
    plsc.subcore_barrier()

    base = jnp.minimum(wid * per_w, base_last)

    def off_of(t):
      return base + jnp.minimum(t, t_last) * B3

    def prefetch(par, t):
      pltpu.async_copy(conn_hbm.at[pl.ds(off_of(t), B3)], gidx[par], si[par])

    def step(par, t, first):
      r = off_of(t)
      pltpu.make_async_copy(conn_hbm.at[pl.ds(r, B3)], gidx[par],
                            si[par]).wait()
      if not first:
        pltpu.make_async_copy(grows[par].at[:, pl.ds(0, 3)],
                              out_hbm.at[pl.ds(r, B3)], so[par]).wait()
      pltpu.sync_copy(table.at[gidx[par]], grows[par])
      pltpu.async_copy(grows[par].at[:, pl.ds(0, 3)],
                       out_hbm.at[pl.ds(r, B3)], so[par])
      prefetch(par, t + 2)

    prefetch(0, 0)
    prefetch(1, 1)
    step(0, 0, True)
    step(1, 1, True)

    @pl.loop(1, steps_r // 2)
    def _pipe(k):
      step(0, 2 * k, False)
      step(1, 2 * k + 1, False)

    for par in range(2):
      pltpu.make_async_copy(conn_hbm.at[pl.ds(base, B3)], gidx[par],
                            si[par]).wait()
      pltpu.make_async_copy(grows[par].at[:, pl.ds(0, 3)],
                            out_hbm.at[pl.ds(base, B3)], so[par]).wait()

  return body(values_reduced, imposed_full, free_idx, conn_flat)


def kernel(values_reduced, imposed_full, free_idx, conn):
  n_nodes = imposed_full.shape[0]
  n_free = values_reduced.shape[0]
  n_elem, npe = conn.shape
  n_flat = n_elem * npe
  out = _field_sc(values_reduced, imposed_full, free_idx,
                  conn.reshape(-1), n_nodes, n_free, n_flat)
  return out.reshape(n_elem, npe, 3)


# speedup vs baseline: 16.2622x; 16.2622x over previous
import jax
import jax.numpy as jnp
from jax import lax
from jax.experimental import pallas as pl
from jax.experimental.pallas import tpu as pltpu
from jax.experimental.pallas import tpu_sc as plsc


def _field_sc(values_reduced, imposed_full, free_idx, conn_l,
              n_nodes, n_free, n_flat):
  info = plsc.get_sparse_core_info()
  nc, ns = info.num_cores, info.num_subcores
  nw = nc * ns

  C1 = 896
  n1 = -(-n_nodes // C1)
  s1 = -(-n1 // ns)
  r1_last = n_nodes - C1

  nb2 = -(-n_free // 128)
  s2 = -(-nb2 // ns)
  off2_last = n_free - 128

  B3 = 2048
  per_w = -(-n_flat // nw)
  per_w = -(-per_w // B3) * B3
  base_last = n_flat - per_w
  steps = per_w // B3
  t_last = steps - 1
  steps_r = -(-steps // 2) * 2

  mesh = plsc.VectorSubcoreMesh(core_axis_name="c", subcore_axis_name="s")

  @pl.kernel(
      out_type=jax.ShapeDtypeStruct((3, n_flat), jnp.float32),
      mesh=mesh,
      compiler_params=pltpu.CompilerParams(use_tc_tiling_on_sc=False,
                                           needs_layout_passes=False),
      scratch_types=[
          pltpu.VMEM_SHARED((n_nodes, 8), jnp.float32),
          pltpu.VMEM((C1, 8), jnp.float32),
          pltpu.VMEM((1, 128), jnp.int32),
          pltpu.VMEM((128, 8), jnp.float32),
          [pltpu.VMEM((B3,), jnp.int32)] * 2,
          pltpu.VMEM((B3, 8), jnp.float32),
          [pltpu.VMEM((3, B3), jnp.float32)] * 2,
          [pltpu.SemaphoreType.DMA] * 2,
          [pltpu.SemaphoreType.DMA] * 2,
          pltpu.SemaphoreType.DMA,
      ],
  )
  def body(vals_hbm, imp_hbm, free_hbm, conn_hbm, out_hbm,
           table, stage, sidx, svals, gidx, grows, obuf, si, so, sg):
    cid = lax.axis_index("c")
    sid = lax.axis_index("s")
    wid = sid * nc + cid

    @pl.loop(0, s1)
    def _copy(i):
      c = sid + i * ns
      r0 = jnp.minimum(c * C1, r1_last)
      pltpu.sync_copy(imp_hbm.at[pl.ds(r0, C1)], stage.at[:, pl.ds(0, 3)])
      pltpu.sync_copy(stage, table.at[pl.ds(r0, C1)])
    plsc.subcore_barrier()

    @pl.loop(0, s2)
    def _scatter(k):
      b = sid + k * ns
      off = jnp.minimum(b * 128, off2_last)
      pltpu.sync_copy(free_hbm.at[pl.ds(off, 128)], sidx.at[0])
      pltpu.sync_copy(vals_hbm.at[pl.ds(off, 128)], svals.at[:, pl.ds(0, 3)])
      pltpu.sync_copy(svals, table.at[sidx.at[0]])
    plsc.subcore_barrier()

    base = jnp.minimum(wid * per_w, base_last)
    lanes = lax.iota(jnp.int32, 16)

    def off_of(t):
      return base + jnp.minimum(t, t_last) * B3

    def prefetch(par, t):
      pltpu.async_copy(conn_hbm.at[pl.ds(off_of(t), B3)], gidx[par], si[par])

    def step(par, t, first):
      r = off_of(t)
      pltpu.make_async_copy(conn_hbm.at[pl.ds(r, B3)], gidx[par],
                            si[par]).wait()
      if not first:
        for c in range(3):
          pltpu.make_async_copy(obuf[par].at[c],
                                out_hbm.at[c, pl.ds(r, B3)], so[par]).wait()
      pltpu.async_copy(table.at[gidx[par]], grows, sg).wait()

      @pl.loop(0, B3 // 16)
      def _extract(g):
        rows = g * 16 + lanes
        for c in range(3):
          v = plsc.load_gather(grows, [rows, jnp.full((16,), c, jnp.int32)])
          obuf[par][c, pl.ds(g * 16, 16)] = v

      for c in range(3):
        pltpu.async_copy(obuf[par].at[c], out_hbm.at[c, pl.ds(r, B3)],
                         so[par])
      prefetch(par, t + 2)

    prefetch(0, 0)
    prefetch(1, 1)
    step(0, 0, True)
    step(1, 1, True)

    @pl.loop(1, steps_r // 2)
    def _pipe(k):
      step(0, 2 * k, False)
      step(1, 2 * k + 1, False)

    for par in range(2):
      pltpu.make_async_copy(conn_hbm.at[pl.ds(base, B3)], gidx[par],
                            si[par]).wait()
      for c in range(3):
        pltpu.make_async_copy(obuf[par].at[c],
                              out_hbm.at[c, pl.ds(base, B3)], so[par]).wait()

  return body(values_reduced, imposed_full, free_idx, conn_l)


def kernel(values_reduced, imposed_full, free_idx, conn):
  n_nodes = imposed_full.shape[0]
  n_free = values_reduced.shape[0]
  n_elem, npe = conn.shape
  n_flat = n_elem * npe
  eb = n_elem // 128
  conn_l = conn.reshape(eb, 128, npe).transpose(0, 2, 1).reshape(-1)
  planes = _field_sc(values_reduced, imposed_full, free_idx, conn_l,
                     n_nodes, n_free, n_flat)
  out = planes.reshape(3, eb, npe, 128).transpose(1, 3, 2, 0)
  return out.reshape(n_elem, npe, 3)


# speedup vs baseline: 27.2767x; 1.6773x over previous
import jax
import jax.numpy as jnp
from jax import lax
from jax.experimental import pallas as pl
from jax.experimental.pallas import tpu as pltpu
from jax.experimental.pallas import tpu_sc as plsc


def _field_sc(vals_t, imp_t, free_idx, conn_l, n_nodes, n_free, n_flat):
  info = plsc.get_sparse_core_info()
  nc, ns = info.num_cores, info.num_subcores
  nw = nc * ns

  C1 = -(-n_nodes // ns)
  C1 = -(-C1 // 8) * 8
  r1_last = n_nodes - C1

  nb2 = -(-n_free // 128)
  s2 = -(-nb2 // ns)
  off2_last = n_free - 128

  B3 = 2048
  per_w = -(-n_flat // nw)
  per_w = -(-per_w // B3) * B3
  base_last = n_flat - per_w
  steps = per_w // B3
  t_last = steps - 1
  steps_r = -(-steps // 2) * 2

  mesh = plsc.VectorSubcoreMesh(core_axis_name="c", subcore_axis_name="s")

  @pl.kernel(
      out_type=jax.ShapeDtypeStruct((3, n_flat), jnp.float32),
      mesh=mesh,
      compiler_params=pltpu.CompilerParams(use_tc_tiling_on_sc=False),
      scratch_types=[
          pltpu.VMEM_SHARED((3, n_nodes), jnp.float32),
          pltpu.VMEM((C1,), jnp.float32),
          pltpu.VMEM((1, 128), jnp.int32),
          [pltpu.VMEM((128,), jnp.float32)] * 3,
          [pltpu.VMEM((B3,), jnp.int32)] * 2,
          [pltpu.VMEM((3, B3), jnp.float32)] * 2,
          [pltpu.SemaphoreType.DMA] * 2,
          [pltpu.SemaphoreType.DMA] * 2,
          [pltpu.SemaphoreType.DMA] * 2,
      ],
  )
  def body(vals_hbm, imp_hbm, free_hbm, conn_hbm, out_hbm,
           table, stage, sidx, svals, gidx, obuf, si, so, sg):
    cid = lax.axis_index("c")
    sid = lax.axis_index("s")
    wid = sid * nc + cid

    base = jnp.minimum(wid * per_w, base_last)

    def off_of(t):
      return base + jnp.minimum(t, t_last) * B3

    def prefetch(par, t):
      pltpu.async_copy(conn_hbm.at[pl.ds(off_of(t), B3)], gidx[par], si[par])

    prefetch(0, 0)
    prefetch(1, 1)

    r0 = jnp.minimum(sid * C1, r1_last)
    for c in range(3):
      pltpu.sync_copy(imp_hbm.at[c, pl.ds(r0, C1)], stage)
      pltpu.sync_copy(stage, table.at[c, pl.ds(r0, C1)])
    plsc.subcore_barrier()

    @pl.loop(0, s2)
    def _scatter(k):
      b = sid + k * ns
      off = jnp.minimum(b * 128, off2_last)
      pltpu.sync_copy(free_hbm.at[pl.ds(off, 128)], sidx.at[0])
      for c in range(3):
        pltpu.sync_copy(vals_hbm.at[c, pl.ds(off, 128)], svals[c])
      for c in range(3):
        pltpu.sync_copy(svals[c], table.at[c].at[sidx.at[0]])
    plsc.subcore_barrier()

    def step(par, t, first):
      r = off_of(t)
      pltpu.make_async_copy(conn_hbm.at[pl.ds(r, B3)], gidx[par],
                            si[par]).wait()
      if not first:
        for c in range(3):
          pltpu.make_async_copy(obuf[par].at[c],
                                out_hbm.at[c, pl.ds(r, B3)], so[par]).wait()
      for c in range(3):
        pltpu.async_copy(table.at[c].at[gidx[par]], obuf[par].at[c],
                         sg[par])
      for c in range(3):
        pltpu.make_async_copy(table.at[c].at[gidx[par]], obuf[par].at[c],
                              sg[par]).wait()
      for c in range(3):
        pltpu.async_copy(obuf[par].at[c], out_hbm.at[c, pl.ds(r, B3)],
                         so[par])
      prefetch(par, t + 2)

    step(0, 0, True)
    step(1, 1, True)

    @pl.loop(1, steps_r // 2)
    def _pipe(k):
      step(0, 2 * k, False)
      step(1, 2 * k + 1, False)

    for par in range(2):
      pltpu.make_async_copy(conn_hbm.at[pl.ds(base, B3)], gidx[par],
                            si[par]).wait()
      for c in range(3):
        pltpu.make_async_copy(obuf[par].at[c],
                              out_hbm.at[c, pl.ds(base, B3)], so[par]).wait()

  return body(vals_t, imp_t, free_idx, conn_l)


def kernel(values_reduced, imposed_full, free_idx, conn):
  n_nodes = imposed_full.shape[0]
  n_free = values_reduced.shape[0]
  n_elem, npe = conn.shape
  n_flat = n_elem * npe
  eb = n_elem // 128
  conn_l = conn.reshape(eb, 128, npe).transpose(0, 2, 1).reshape(-1)
  planes = _field_sc(values_reduced.T, imposed_full.T, free_idx, conn_l,
                     n_nodes, n_free, n_flat)
  out = planes.reshape(3, eb, npe, 128).transpose(1, 3, 2, 0)
  return out.reshape(n_elem, npe, 3)


# speedup vs baseline: 38.1908x; 1.4001x over previous
import jax
import jax.numpy as jnp
from jax import lax
from jax.experimental import pallas as pl
from jax.experimental.pallas import tpu as pltpu
from jax.experimental.pallas import tpu_sc as plsc


def _field_sc(vals_t, imp_t, free_idx, conn_l, n_nodes, n_free, n_flat):
  info = plsc.get_sparse_core_info()
  nc, ns = info.num_cores, info.num_subcores
  nw = nc * ns

  C1 = -(-n_nodes // ns)
  C1 = -(-C1 // 8) * 8
  r1_last = n_nodes - C1

  nb2 = -(-n_free // 128)
  s2 = -(-nb2 // ns)
  off2_last = n_free - 128

  B3 = 4096
  per_w = -(-n_flat // nw)
  per_w = -(-per_w // B3) * B3
  base_last = n_flat - per_w
  steps = per_w // B3
  t_last = steps - 1
  steps_r = -(-steps // 2) * 2

  mesh = plsc.VectorSubcoreMesh(core_axis_name="c", subcore_axis_name="s")

  @pl.kernel(
      out_type=jax.ShapeDtypeStruct((3, n_flat), jnp.float32),
      mesh=mesh,
      compiler_params=pltpu.CompilerParams(use_tc_tiling_on_sc=False),
      scratch_types=[
          pltpu.VMEM_SHARED((3, n_nodes), jnp.float32),
          pltpu.VMEM((C1,), jnp.float32),
          pltpu.VMEM((s2, 128), jnp.int32),
          pltpu.VMEM((s2, 3, 128), jnp.float32),
          [pltpu.VMEM((B3,), jnp.int32)] * 2,
          [pltpu.VMEM((3, B3), jnp.float32)] * 2,
          [pltpu.SemaphoreType.DMA] * 2,
          [pltpu.SemaphoreType.DMA] * 2,
          [pltpu.SemaphoreType.DMA] * 2,
          pltpu.SemaphoreType.DMA,
          pltpu.SemaphoreType.DMA,
      ],
  )
  def body(vals_hbm, imp_hbm, free_hbm, conn_hbm, out_hbm,
           table, stage, sidx, svals, gidx, obuf, si, so, sg, s2l, s2s):
    cid = lax.axis_index("c")
    sid = lax.axis_index("s")
    wid = sid * nc + cid

    base = jnp.minimum(wid * per_w, base_last)

    def off_of(t):
      return base + jnp.minimum(t, t_last) * B3

    def prefetch(par, t):
      pltpu.async_copy(conn_hbm.at[pl.ds(off_of(t), B3)], gidx[par], si[par])

    prefetch(0, 0)
    prefetch(1, 1)

    def off2_of(k):
      return jnp.minimum((sid * s2 + k) * 128, off2_last)

    @pl.loop(0, s2)
    def _p2load(k):
      off = off2_of(k)
      pltpu.async_copy(free_hbm.at[pl.ds(off, 128)], sidx.at[k], s2l)
      pltpu.async_copy(vals_hbm.at[:, pl.ds(off, 128)], svals.at[k], s2l)

    r0 = jnp.minimum(sid * C1, r1_last)
    for c in range(3):
      pltpu.sync_copy(imp_hbm.at[c, pl.ds(r0, C1)], stage)
      pltpu.sync_copy(stage, table.at[c, pl.ds(r0, C1)])
    plsc.subcore_barrier()

    @pl.loop(0, s2)
    def _p2drain(k):
      off = off2_of(k)
      pltpu.make_async_copy(free_hbm.at[pl.ds(off, 128)], sidx.at[k],
                            s2l).wait()
      pltpu.make_async_copy(vals_hbm.at[:, pl.ds(off, 128)], svals.at[k],
                            s2l).wait()

    @pl.loop(0, s2)
    def _p2scat(k):
      for c in range(3):
        pltpu.async_copy(svals.at[k].at[c], table.at[c].at[sidx.at[k]], s2s)

    @pl.loop(0, s2)
    def _p2wait(k):
      for c in range(3):
        pltpu.make_async_copy(svals.at[k].at[c],
                              table.at[c].at[sidx.at[k]], s2s).wait()
    plsc.subcore_barrier()

    def step(par, t, first):
      r = off_of(t)
      pltpu.make_async_copy(conn_hbm.at[pl.ds(r, B3)], gidx[par],
                            si[par]).wait()
      if not first:
        for c in range(3):
          pltpu.make_async_copy(obuf[par].at[c],
                                out_hbm.at[c, pl.ds(r, B3)], so[par]).wait()
      for c in range(3):
        pltpu.async_copy(table.at[c].at[gidx[par]], obuf[par].at[c],
                         sg[par])
      for c in range(3):
        pltpu.make_async_copy(table.at[c].at[gidx[par]], obuf[par].at[c],
                              sg[par]).wait()
      for c in range(3):
        pltpu.async_copy(obuf[par].at[c], out_hbm.at[c, pl.ds(r, B3)],
                         so[par])
      prefetch(par, t + 2)

    step(0, 0, True)
    step(1, 1, True)

    @pl.loop(1, steps_r // 2)
    def _pipe(k):
      step(0, 2 * k, False)
      step(1, 2 * k + 1, False)

    for par in range(2):
      pltpu.make_async_copy(conn_hbm.at[pl.ds(base, B3)], gidx[par],
                            si[par]).wait()
      for c in range(3):
        pltpu.make_async_copy(obuf[par].at[c],
                              out_hbm.at[c, pl.ds(base, B3)], so[par]).wait()

  return body(vals_t, imp_t, free_idx, conn_l)


def kernel(values_reduced, imposed_full, free_idx, conn):
  n_nodes = imposed_full.shape[0]
  n_free = values_reduced.shape[0]
  n_elem, npe = conn.shape
  n_flat = n_elem * npe
  eb = n_elem // 128
  conn_l = conn.reshape(eb, 128, npe).transpose(0, 2, 1).reshape(-1)
  planes = _field_sc(values_reduced.T, imposed_full.T, free_idx, conn_l,
                     n_nodes, n_free, n_flat)
  out = planes.reshape(3, eb, npe, 128).transpose(1, 3, 2, 0)
  return out.reshape(n_elem, npe, 3)
